# baseline (speedup 1.0000x reference)
import functools

import jax
import jax.numpy as jnp
from jax import lax
from jax.experimental import pallas as pl
from jax.experimental.pallas import tpu as pltpu
from jax.experimental.pallas import tpu_sc as plsc

GENE_DIM = 64
VPAD = 64
HIDDEN = 128
BATCH = 64
SEQ = 2048

_info = plsc.get_sparse_core_info()
_NC, _NS = _info.num_cores, _info.num_subcores
NW = _NC * _NS
SW = SEQ // NW
TPT = BATCH * SW

_sc_mesh = plsc.VectorSubcoreMesh(core_axis_name="c", subcore_axis_name="s")


def _precompute_kernel(gene_ref, expr_ref, w_ref, b_ref, pos_ref,
                       g_out_ref, e_out_ref, m_out_ref, rs_out_ref):
    w1 = w_ref[0:GENE_DIM, :]
    w2 = w_ref[GENE_DIM: 2 * GENE_DIM, :]
    g = jax.lax.dot_general(
        gene_ref[...], w1, (((1,), (0,)), ((), ())),
        preferred_element_type=jnp.float32,
    ) + b_ref[...] + pos_ref[...]
    e = jax.lax.dot_general(
        expr_ref[...], w2, (((1,), (0,)), ((), ())),
        preferred_element_type=jnp.float32,
    )
    g_out_ref[...] = g
    e_out_ref[...] = e
    ones = jnp.ones((1, HIDDEN), dtype=jnp.float32)
    inv_h = 1.0 / HIDDEN
    mg = jnp.sum(g, axis=-1, keepdims=True) * inv_h
    me = jax.lax.dot_general(
        ones, e, (((1,), (1,)), ((), ())),
        preferred_element_type=jnp.float32) * inv_h
    s2g = jnp.sum(g * g, axis=-1, keepdims=True)
    s2e = jax.lax.dot_general(
        ones, e * e, (((1,), (1,)), ((), ())),
        preferred_element_type=jnp.float32)
    cross = jax.lax.dot_general(
        g, e, (((1,), (1,)), ((), ())),
        preferred_element_type=jnp.float32)
    m = mg + me
    q = (s2g + 2.0 * cross + s2e) * inv_h - m * m + 1e-5
    m_out_ref[...] = m
    rs_out_ref[...] = jax.lax.rsqrt(q)


@functools.partial(
    pl.kernel, mesh=_sc_mesh,
    out_type=jax.ShapeDtypeStruct((BATCH, SEQ, HIDDEN), jnp.float32),
    scratch_types=[
        pltpu.VMEM((TPT,), jnp.int32),
        pltpu.VMEM((TPT,), jnp.int32),
        pltpu.VMEM((SW, HIDDEN), jnp.float32),
        pltpu.VMEM((VPAD, HIDDEN), jnp.float32),
        pltpu.VMEM((TPT,), jnp.float32),
        pltpu.VMEM((TPT,), jnp.float32),
        pltpu.VMEM((HIDDEN,), jnp.float32),
        pltpu.VMEM((HIDDEN,), jnp.float32),
        pltpu.VMEM((SW, HIDDEN), jnp.float32),
        pltpu.SemaphoreType.DMA,
    ],
)
def _sc_main(ids_hbm, g_hbm, e_hbm, m_hbm, rs_hbm, gamma_hbm, beta_hbm, out_hbm,
             ids_v, idx_v, g_v, e_v, m_v, rs_v, gamma_v, beta_v, y_v, sem):
    wid = lax.axis_index("s") * _NC + lax.axis_index("c")
    s0 = wid * SW
    pltpu.sync_copy(ids_hbm.at[wid], ids_v)
    pltpu.sync_copy(g_hbm.at[pl.ds(s0, SW)], g_v)
    pltpu.sync_copy(e_hbm, e_v)
    pltpu.sync_copy(gamma_hbm, gamma_v)
    pltpu.sync_copy(beta_hbm, beta_v)
    lane_iota = lax.iota(jnp.int32, 16)

    def idx_body(i, carry):
        ids16 = ids_v[pl.ds(16 * i, 16)]
        r16 = lax.rem(16 * i + lane_iota, SW)
        idx_v[pl.ds(16 * i, 16)] = (s0 + r16) * VPAD + ids16
        return carry

    lax.fori_loop(0, TPT // 16, idx_body, 0)
    pltpu.async_copy(m_hbm.at[idx_v], m_v, sem).wait()
    pltpu.async_copy(rs_hbm.at[idx_v], rs_v, sem).wait()

    gammas = [gamma_v[pl.ds(16 * j, 16)] for j in range(8)]
    betas = [beta_v[pl.ds(16 * j, 16)] for j in range(8)]

    def batch_body(b, carry):
        def group_body(gidx, carry2):
            r0 = gidx * 16
            t0 = b * SW + r0
            idvec = ids_v[pl.ds(t0, 16)]
            rs16 = rs_v[pl.ds(t0, 16)]
            c16 = -(m_v[pl.ds(t0, 16)]) * rs16
            for k in range(16):
                r = r0 + k
                vid = idvec[k]
                rs = rs16[k]
                c = c16[k]
                for j in range(8):
                    x = e_v[vid, pl.ds(16 * j, 16)] + g_v[r, pl.ds(16 * j, 16)]
                    y_v[r, pl.ds(16 * j, 16)] = (x * rs + c) * gammas[j] + betas[j]
            return carry2

        lax.fori_loop(0, SW // 16, group_body, 0)
        pltpu.sync_copy(y_v, out_hbm.at[b, pl.ds(s0, SW)])
        return carry

    lax.fori_loop(0, BATCH, batch_body, 0)


@jax.jit
def kernel(input_ids, gene_table, expr_table, W_proj, b_proj, pos_table, ln_gamma, ln_beta):
    B, S = input_ids.shape
    V, D = expr_table.shape
    H = W_proj.shape[1]

    expr_pad = jnp.zeros((VPAD, D), dtype=expr_table.dtype).at[:V].set(expr_table)

    g_tab, e_tab, m_tab, rs_tab = pl.pallas_call(
        _precompute_kernel,
        grid=(1,),
        in_specs=[
            pl.BlockSpec((S, D), lambda i: (0, 0)),
            pl.BlockSpec((VPAD, D), lambda i: (0, 0)),
            pl.BlockSpec((2 * D, H), lambda i: (0, 0)),
            pl.BlockSpec((1, H), lambda i: (0, 0)),
            pl.BlockSpec((S, H), lambda i: (0, 0)),
        ],
        out_specs=[
            pl.BlockSpec((S, H), lambda i: (0, 0)),
            pl.BlockSpec((VPAD, H), lambda i: (0, 0)),
            pl.BlockSpec((S, VPAD), lambda i: (0, 0)),
            pl.BlockSpec((S, VPAD), lambda i: (0, 0)),
        ],
        out_shape=[
            jax.ShapeDtypeStruct((S, H), jnp.float32),
            jax.ShapeDtypeStruct((VPAD, H), jnp.float32),
            jax.ShapeDtypeStruct((S, VPAD), jnp.float32),
            jax.ShapeDtypeStruct((S, VPAD), jnp.float32),
        ],
    )(gene_table, expr_pad, W_proj, b_proj.reshape(1, H), pos_table)

    ids_t = (input_ids.astype(jnp.int32)
             .reshape(B, NW, SW).transpose(1, 0, 2).reshape(NW, TPT))
    return _sc_main(ids_t, g_tab, e_tab,
                    m_tab.reshape(-1), rs_tab.reshape(-1), ln_gamma, ln_beta)


# speedup vs baseline: 4.3350x; 4.3350x over previous
import functools

import jax
import jax.numpy as jnp
from jax.experimental import pallas as pl
from jax.experimental.pallas import tpu as pltpu

N_GENES = 100000
GENE_DIM = 64
EXPR_PAD = 64
HIDDEN = 128
BATCH = 64
SEQ = 2048
TOK_BLOCK = 8192


def _precompute_kernel(gene_ref, expr_ref, w_ref, b_ref, pos_ref, g_out_ref, e_out_ref):
    w1 = w_ref[0:GENE_DIM, :]
    w2 = w_ref[GENE_DIM : 2 * GENE_DIM, :]
    g = jax.lax.dot_general(
        gene_ref[...], w1, (((1,), (0,)), ((), ())),
        preferred_element_type=jnp.float32,
    )
    g_out_ref[...] = g + b_ref[...] + pos_ref[...]
    e_out_ref[...] = jax.lax.dot_general(
        expr_ref[...], w2, (((1,), (0,)), ((), ())),
        preferred_element_type=jnp.float32,
    )


def _main_kernel(ids_ref, g_ref, e_ref, gamma_ref, beta_ref, out_ref):
    ids = ids_ref[0, :, :]
    ids_b = jnp.broadcast_to(ids, (EXPR_PAD, TOK_BLOCK))
    vocab_iota = jax.lax.broadcasted_iota(jnp.int32, (EXPR_PAD, TOK_BLOCK), 0)
    onehot = (ids_b == vocab_iota).astype(jnp.float32)
    gathered = jax.lax.dot_general(
        onehot, e_ref[...], (((0,), (0,)), ((), ())),
        preferred_element_type=jnp.float32,
    )
    k = TOK_BLOCK // SEQ
    x = gathered.reshape(k, SEQ, HIDDEN) + g_ref[...][None, :, :]
    x = x.reshape(TOK_BLOCK, HIDDEN)
    mean = jnp.mean(x, axis=-1, keepdims=True)
    centered = x - mean
    var = jnp.mean(centered * centered, axis=-1, keepdims=True)
    y = centered * jax.lax.rsqrt(var + 1e-5)
    out_ref[...] = y * gamma_ref[...] + beta_ref[...]


@jax.jit
def kernel(input_ids, gene_table, expr_table, W_proj, b_proj, pos_table, ln_gamma, ln_beta):
    B, S = input_ids.shape
    V, D = expr_table.shape
    H = W_proj.shape[1]

    expr_pad = jnp.zeros((EXPR_PAD, D), dtype=expr_table.dtype).at[:V].set(expr_table)

    g_tab, e_tab = pl.pallas_call(
        _precompute_kernel,
        grid=(1,),
        in_specs=[
            pl.BlockSpec((S, D), lambda i: (0, 0)),
            pl.BlockSpec((EXPR_PAD, D), lambda i: (0, 0)),
            pl.BlockSpec((2 * D, H), lambda i: (0, 0)),
            pl.BlockSpec((1, H), lambda i: (0, 0)),
            pl.BlockSpec((S, H), lambda i: (0, 0)),
        ],
        out_specs=[
            pl.BlockSpec((S, H), lambda i: (0, 0)),
            pl.BlockSpec((EXPR_PAD, H), lambda i: (0, 0)),
        ],
        out_shape=[
            jax.ShapeDtypeStruct((S, H), jnp.float32),
            jax.ShapeDtypeStruct((EXPR_PAD, H), jnp.float32),
        ],
    )(gene_table, expr_pad, W_proj, b_proj.reshape(1, H), pos_table)

    n_blocks = (B * S) // TOK_BLOCK
    ids3 = input_ids.astype(jnp.int32).reshape(n_blocks, 1, TOK_BLOCK)

    out_flat = pl.pallas_call(
        _main_kernel,
        grid=(n_blocks,),
        in_specs=[
            pl.BlockSpec((1, 1, TOK_BLOCK), lambda i: (i, 0, 0)),
            pl.BlockSpec((S, H), lambda i: (0, 0)),
            pl.BlockSpec((EXPR_PAD, H), lambda i: (0, 0)),
            pl.BlockSpec((1, H), lambda i: (0, 0)),
            pl.BlockSpec((1, H), lambda i: (0, 0)),
        ],
        out_specs=pl.BlockSpec((TOK_BLOCK, H), lambda i: (i, 0)),
        out_shape=jax.ShapeDtypeStruct((B * S, H), jnp.float32),
        compiler_params=pltpu.CompilerParams(
            dimension_semantics=("parallel",),
        ),
    )(ids3, g_tab, e_tab, ln_gamma.reshape(1, H), ln_beta.reshape(1, H))

    return out_flat.reshape(B, S, H)


# speedup vs baseline: 4.4168x; 1.0189x over previous
import functools

import jax
import jax.numpy as jnp
from jax.experimental import pallas as pl
from jax.experimental.pallas import tpu as pltpu

N_GENES = 100000
GENE_DIM = 64
EXPR_PAD = 64
HIDDEN = 128
BATCH = 64
SEQ = 2048
TOK_BLOCK = 8192


def _fused_kernel(ids_ref, gene_ref, expr_ref, w_ref, b_ref, pos_ref,
                  gamma_ref, beta_ref, out_ref, g_ref, e_ref):
    @pl.when(pl.program_id(0) == 0)
    def _():
        w1 = w_ref[0:GENE_DIM, :]
        w2 = w_ref[GENE_DIM : 2 * GENE_DIM, :]
        g = jax.lax.dot_general(
            gene_ref[...], w1, (((1,), (0,)), ((), ())),
            preferred_element_type=jnp.float32,
        )
        g_ref[...] = g + b_ref[...] + pos_ref[...]
        e_ref[...] = jax.lax.dot_general(
            expr_ref[...], w2, (((1,), (0,)), ((), ())),
            preferred_element_type=jnp.float32,
        )

    ids = ids_ref[0, :, :]
    ids_b = jnp.broadcast_to(ids, (EXPR_PAD, TOK_BLOCK))
    vocab_iota = jax.lax.broadcasted_iota(jnp.int32, (EXPR_PAD, TOK_BLOCK), 0)
    onehot = (ids_b == vocab_iota).astype(jnp.float32)
    gathered = jax.lax.dot_general(
        onehot, e_ref[...], (((0,), (0,)), ((), ())),
        preferred_element_type=jnp.float32,
    )
    k = TOK_BLOCK // SEQ
    x = gathered.reshape(k, SEQ, HIDDEN) + g_ref[...][None, :, :]
    x = x.reshape(TOK_BLOCK, HIDDEN)
    mean = jnp.mean(x, axis=-1, keepdims=True)
    centered = x - mean
    var = jnp.mean(centered * centered, axis=-1, keepdims=True)
    y = centered * jax.lax.rsqrt(var + 1e-5)
    out_ref[...] = y * gamma_ref[...] + beta_ref[...]


@jax.jit
def kernel(input_ids, gene_table, expr_table, W_proj, b_proj, pos_table, ln_gamma, ln_beta):
    B, S = input_ids.shape
    V, D = expr_table.shape
    H = W_proj.shape[1]

    expr_pad = jnp.zeros((EXPR_PAD, D), dtype=expr_table.dtype).at[:V].set(expr_table)

    n_blocks = (B * S) // TOK_BLOCK
    ids3 = input_ids.astype(jnp.int32).reshape(n_blocks, 1, TOK_BLOCK)

    out_flat = pl.pallas_call(
        _fused_kernel,
        grid=(n_blocks,),
        in_specs=[
            pl.BlockSpec((1, 1, TOK_BLOCK), lambda i: (i, 0, 0)),
            pl.BlockSpec((S, D), lambda i: (0, 0)),
            pl.BlockSpec((EXPR_PAD, D), lambda i: (0, 0)),
            pl.BlockSpec((2 * D, H), lambda i: (0, 0)),
            pl.BlockSpec((1, H), lambda i: (0, 0)),
            pl.BlockSpec((S, H), lambda i: (0, 0)),
            pl.BlockSpec((1, H), lambda i: (0, 0)),
            pl.BlockSpec((1, H), lambda i: (0, 0)),
        ],
        out_specs=pl.BlockSpec((TOK_BLOCK, H), lambda i: (i, 0)),
        out_shape=jax.ShapeDtypeStruct((B * S, H), jnp.float32),
        scratch_shapes=[
            pltpu.VMEM((S, H), jnp.float32),
            pltpu.VMEM((EXPR_PAD, H), jnp.float32),
        ],
    )(ids3, gene_table, expr_pad, W_proj, b_proj.reshape(1, H), pos_table,
      ln_gamma.reshape(1, H), ln_beta.reshape(1, H))

    return out_flat.reshape(B, S, H)
